# Initial kernel scaffold; baseline (speedup 1.0000x reference)
import jax
import jax.numpy as jnp
from jax.experimental import pallas as pl


def kernel(x, y):
    raise NotImplementedError("write your pallas kernel here")


# speedup vs baseline: 25.4747x; 25.4747x over previous
import functools

import numpy as np
import jax
import jax.numpy as jnp
from jax import lax
from jax.experimental import pallas as pl
from jax.experimental.pallas import tpu as pltpu
from jax.experimental.pallas import tpu_sc as plsc


HARD_P = 0.5
RAND_P = 0.1


def _threefry2x32(k0, k1, x0, x1):
    rotations = ((13, 15, 26, 6), (17, 29, 16, 24))
    ks = (np.uint32(k0), np.uint32(k1),
          np.uint32(k0) ^ np.uint32(k1) ^ np.uint32(0x1BD11BDA))
    x0 = (x0 + ks[0]).astype(np.uint32)
    x1 = (x1 + ks[1]).astype(np.uint32)
    for i in range(5):
        for r in rotations[i % 2]:
            x0 = (x0 + x1).astype(np.uint32)
            x1 = ((x1 << np.uint32(r)) | (x1 >> np.uint32(32 - r))) ^ x0
        x0 = (x0 + ks[(i + 1) % 3]).astype(np.uint32)
        x1 = (x1 + ks[(i + 2) % 3] + np.uint32(i + 1)).astype(np.uint32)
    return x0, x1


def _threefry_bits(k0, k1, size):
    hi = np.zeros(size, dtype=np.uint32)
    lo = np.arange(size, dtype=np.uint32)
    x0, x1 = _threefry2x32(k0, k1, hi, lo)
    return x0 ^ x1


def _key_split(k0, k1, num):
    hi = np.zeros(num, dtype=np.uint32)
    lo = np.arange(num, dtype=np.uint32)
    b0, b1 = _threefry2x32(k0, k1, hi, lo)
    return np.stack([b0, b1], axis=1)


@functools.lru_cache(maxsize=None)
def _random_tag_np(b: int, h: int, w: int) -> np.ndarray:
    n = h * w
    rand_k = int(RAND_P * n)
    row_keys = _key_split(0, 1, b)
    num_rounds = int(np.ceil(3 * np.log(max(2, n)) / np.log(2**32 - 1)))
    mask = np.zeros((b, n), dtype=np.uint32)
    for i in range(b):
        k0, k1 = row_keys[i]
        perm = np.arange(n, dtype=np.int32)
        for _ in range(num_rounds):
            subs = _key_split(k0, k1, 2)
            (k0, k1), (s0, s1) = subs[0], subs[1]
            sort_keys = _threefry_bits(s0, s1, n)
            perm = perm[np.argsort(sort_keys, kind="stable")]
        mask[i] = perm < rand_k
    return (mask << np.uint32(31)).view(np.int32).reshape(b, h, w)


def _res_body(x_ref, y_ref, rt_ref, out_ref):
    d = jnp.abs(x_ref[0] - y_ref[0])
    res = jnp.sum(d, axis=0)
    out_ref[0] = lax.bitcast_convert_type(res, jnp.int32) | rt_ref[0]


def _residual_tagged(x, y, rtag):
    b, c, h, w = x.shape
    return pl.pallas_call(
        _res_body,
        grid=(b,),
        in_specs=[
            pl.BlockSpec((1, c, h, w), lambda i: (i, 0, 0, 0)),
            pl.BlockSpec((1, c, h, w), lambda i: (i, 0, 0, 0)),
            pl.BlockSpec((1, h, w), lambda i: (i, 0, 0)),
        ],
        out_specs=pl.BlockSpec((1, h, w), lambda i: (i, 0, 0)),
        out_shape=jax.ShapeDtypeStruct((b, h, w), jnp.int32),
    )(x, y, rtag)


_ROUNDS = ((23, 8), (15, 8), (7, 8), (0, 7))
_NBMAX = 256


def _make_select(b, n, hard_k):
    info = plsc.get_sparse_core_info()
    ncores, nsub = info.num_cores, info.num_subcores
    rows_per_core = b // ncores
    wpr = nsub // rows_per_core
    qwords = n // wpr
    mesh = plsc.VectorSubcoreMesh(core_axis_name="c", subcore_axis_name="s")

    @functools.partial(
        pl.kernel,
        out_type=jax.ShapeDtypeStruct((b, 16), jnp.int32),
        mesh=mesh,
        compiler_params=pltpu.CompilerParams(needs_layout_passes=False),
        scratch_types=[
            pltpu.VMEM((qwords,), jnp.int32),
            pltpu.VMEM((_NBMAX * 16,), jnp.int32),
            pltpu.VMEM((wpr, _NBMAX * 16), jnp.int32),
            pltpu.VMEM((16,), jnp.int32),
            pltpu.VMEM_SHARED((rows_per_core, wpr, _NBMAX * 16), jnp.int32),
        ],
    )
    def select(rt_hbm, out_hbm, data, hist, scanb, obuf, sh_hist):
        s = lax.axis_index("s")
        row_local = s // wpr
        q = s % wpr
        row = lax.axis_index("c") * rows_per_core + row_local
        lane = lax.iota(jnp.int32, 16)
        ones = jnp.ones((16,), jnp.int32)

        pltpu.sync_copy(rt_hbm.at[row, pl.ds(q * qwords, qwords)], data)

        pfx = jnp.int32(0)
        kk = jnp.int32(hard_k)
        done_bits = 0
        for (shift, rbits) in _ROUNDS:
            nb = 1 << rbits
            nwords = nb * 16

            def zero_body(i, _):
                hist[pl.ds(i * 16, 16)] = jnp.zeros((16,), jnp.int32)
                return 0

            lax.fori_loop(0, nb, zero_body, 0)

            pfx_vec = jnp.broadcast_to(pfx, (16,))

            def data_body(j, _):
                t = data[pl.ds(j * 16, 16)]
                vb = t & jnp.int32(0x7FFFFFFF)
                bin_ = (vb >> shift) & jnp.int32(nb - 1)
                addr = bin_ * 16 + lane
                if done_bits == 0:
                    plsc.addupdate_scatter(hist, [addr], ones)
                else:
                    ok = (vb >> (shift + rbits)) == pfx_vec
                    plsc.addupdate_scatter(hist, [addr], ones, mask=ok)
                return 0

            lax.fori_loop(0, qwords // 16, data_body, 0)

            pltpu.sync_copy(hist.at[pl.ds(0, nwords)],
                            sh_hist.at[row_local, q, pl.ds(0, nwords)])
            plsc.subcore_barrier()
            pltpu.sync_copy(sh_hist.at[row_local], scanb)
            plsc.subcore_barrier()

            ngroups = nb // 16

            def gscan(i, carry):
                g = ngroups - 1 - i
                running, found, gstar, above = carry

                def accw(wi, acc):
                    def accv(v, acc2):
                        return acc2 + scanb[wi, pl.ds(g * 256 + v * 16, 16)]
                    return lax.fori_loop(0, 16, accv, acc)

                acc = lax.fori_loop(0, wpr, accw,
                                    jnp.zeros((16,), jnp.int32))
                gsum = jnp.sum(acc)
                cross = jnp.logical_and(found == 0, running + gsum >= kk + 1)
                gstar = jnp.where(cross, g, gstar)
                above = jnp.where(cross, running, above)
                found = jnp.where(cross, 1, found)
                return (running + gsum, found, gstar, above)

            init = (jnp.int32(0), jnp.int32(0), jnp.int32(0), jnp.int32(0))
            _, _, gstar, above_g = lax.fori_loop(0, ngroups, gscan, init)

            def bscan(i, carry):
                bin_ = gstar * 16 + (15 - i)
                running, found, bstar, above = carry

                def accw(wi, acc):
                    return acc + scanb[wi, pl.ds(bin_ * 16, 16)]

                acc = lax.fori_loop(0, wpr, accw,
                                    jnp.zeros((16,), jnp.int32))
                bsum = jnp.sum(acc)
                cross = jnp.logical_and(found == 0, running + bsum >= kk + 1)
                bstar = jnp.where(cross, bin_, bstar)
                above = jnp.where(cross, running, above)
                found = jnp.where(cross, 1, found)
                return (running + bsum, found, bstar, above)

            init2 = (above_g, jnp.int32(0), jnp.int32(0), above_g)
            _, _, bstar, above_b = lax.fori_loop(0, 16, bscan, init2)

            pfx = (pfx << rbits) | bstar
            kk = kk - above_b
            done_bits += rbits

        @pl.when(q == 0)
        def _write_out():
            obuf[...] = jnp.broadcast_to(pfx, (16,))
            pltpu.sync_copy(obuf, out_hbm.at[row])

    return select


def _sum_body(rt_ref, thre_ref, out_ref):
    i = pl.program_id(0)
    t = rt_ref[0]
    vb = t & jnp.int32(0x7FFFFFFF)
    res = lax.bitcast_convert_type(vb, jnp.float32)
    keep = (vb > thre_ref[i]) | (t < 0)
    contrib = jnp.sum(jnp.where(keep, res, 0.0))

    @pl.when(i == 0)
    def _():
        out_ref[0, 0] = 0.0

    out_ref[0, 0] += contrib


def _masked_sum(rt, thre):
    b, h, w = rt.shape
    return pl.pallas_call(
        _sum_body,
        grid=(b,),
        in_specs=[
            pl.BlockSpec((1, h, w), lambda i: (i, 0, 0)),
            pl.BlockSpec(memory_space=pltpu.SMEM),
        ],
        out_specs=pl.BlockSpec(memory_space=pltpu.SMEM),
        out_shape=jax.ShapeDtypeStruct((1, 1), jnp.float32),
    )(rt, thre)


def kernel(x, y):
    b, c, h, w = x.shape
    n = h * w
    hard_k = int(HARD_P * n)
    rtag = jnp.asarray(_random_tag_np(b, h, w))
    rt = _residual_tagged(x, y, rtag)
    thre16 = _make_select(b, n, hard_k)(rt.reshape(b, n))
    out = _masked_sum(rt, thre16[:, 0])
    return out[0, 0] / (b * c * h * w)
